# baseline (device time: 62291 ns/iter reference)
import jax
import jax.numpy as jnp
from jax import lax
from jax.experimental import pallas as pl
from jax.experimental.pallas import tpu as pltpu

C = 16


def kernel(x):
    m, n = x.shape
    mc = m // C

    def body(x_ref, out_ref, xstage_ref, xbf_ref, xrecv_ref, red_ref,
             gath_ref, sem_in, sem1s, sem1r, sem2s, sem2r, sem_o1, sem_o2):
        mx = lax.axis_index("x")
        my = lax.axis_index("y")

        in_cp = []
        for c in range(C):
            rows = pl.ds(c * mc, mc)
            cp = pltpu.make_async_copy(
                x_ref.at[rows], xstage_ref.at[rows], sem_in.at[c])
            cp.start()
            in_cp.append(cp)

        barrier = pltpu.get_barrier_semaphore()
        pl.semaphore_signal(barrier, inc=1, device_id=(1 - mx, my),
                            device_id_type=pl.DeviceIdType.MESH)
        pl.semaphore_signal(barrier, inc=1, device_id=(mx, 1 - my),
                            device_id_type=pl.DeviceIdType.MESH)
        pl.semaphore_wait(barrier, 2)

        rdma1 = []
        for c in range(C):
            rows = pl.ds(c * mc, mc)
            in_cp[c].wait()
            xbf_ref[rows, :] = xstage_ref[rows, :].astype(jnp.bfloat16)
            r = pltpu.make_async_remote_copy(
                src_ref=xbf_ref.at[rows],
                dst_ref=xrecv_ref.at[rows],
                send_sem=sem1s.at[c],
                recv_sem=sem1r.at[c],
                device_id=(1 - mx, my),
                device_id_type=pl.DeviceIdType.MESH,
            )
            r.start()
            rdma1.append(r)

        rdma2 = []
        o1_cp = []
        for c in range(C):
            rows = pl.ds(c * mc, mc)
            rdma1[c].wait_recv()
            red_ref[rows, :] = xbf_ref[rows, :] + xrecv_ref[rows, :]
            r = pltpu.make_async_remote_copy(
                src_ref=red_ref.at[rows],
                dst_ref=gath_ref.at[rows],
                send_sem=sem2s.at[c],
                recv_sem=sem2r.at[c],
                device_id=(mx, 1 - my),
                device_id_type=pl.DeviceIdType.MESH,
            )
            r.start()
            rdma2.append(r)
            cp = pltpu.make_async_copy(
                red_ref.at[rows],
                out_ref.at[rows, pl.ds(my * n, n)],
                sem_o1.at[c],
            )
            cp.start()
            o1_cp.append(cp)

        o2_cp = []
        for c in range(C):
            rows = pl.ds(c * mc, mc)
            rdma2[c].wait_recv()
            cp = pltpu.make_async_copy(
                gath_ref.at[rows],
                out_ref.at[rows, pl.ds((1 - my) * n, n)],
                sem_o2.at[c],
            )
            cp.start()
            o2_cp.append(cp)

        for c in range(C):
            rdma1[c].wait_send()
            rdma2[c].wait_send()
            o1_cp[c].wait()
            o2_cp[c].wait()

    return pl.pallas_call(
        body,
        out_shape=jax.ShapeDtypeStruct((m, 2 * n), jnp.bfloat16),
        in_specs=[pl.BlockSpec(memory_space=pl.ANY)],
        out_specs=pl.BlockSpec(memory_space=pl.ANY),
        scratch_shapes=[
            pltpu.VMEM((m, n), jnp.float32),
            pltpu.VMEM((m, n), jnp.bfloat16),
            pltpu.VMEM((m, n), jnp.bfloat16),
            pltpu.VMEM((m, n), jnp.bfloat16),
            pltpu.VMEM((m, n), jnp.bfloat16),
            pltpu.SemaphoreType.DMA((C,)),
            pltpu.SemaphoreType.DMA((C,)),
            pltpu.SemaphoreType.DMA((C,)),
            pltpu.SemaphoreType.DMA((C,)),
            pltpu.SemaphoreType.DMA((C,)),
            pltpu.SemaphoreType.DMA((C,)),
            pltpu.SemaphoreType.DMA((C,)),
        ],
        compiler_params=pltpu.CompilerParams(collective_id=0),
    )(x)


# device time: 60274 ns/iter; 1.0335x vs baseline; 1.0335x over previous
import jax
import jax.numpy as jnp
from jax import lax
from jax.experimental import pallas as pl
from jax.experimental.pallas import tpu as pltpu

C = 32


def kernel(x):
    m, n = x.shape
    mc = m // C

    def body(x_ref, out_ref, xbf_ref, xrecv_ref,
             sem1s, sem1r, sem2s, sem2r):
        mx = lax.axis_index("x")
        my = lax.axis_index("y")

        barrier = pltpu.get_barrier_semaphore()
        pl.semaphore_signal(barrier, inc=1, device_id=(1 - mx, my),
                            device_id_type=pl.DeviceIdType.MESH)
        pl.semaphore_signal(barrier, inc=1, device_id=(mx, 1 - my),
                            device_id_type=pl.DeviceIdType.MESH)
        pl.semaphore_wait(barrier, 2)

        rdma1 = []
        for c in range(C):
            rows = pl.ds(c * mc, mc)
            xbf_ref[rows, :] = x_ref[rows, :].astype(jnp.bfloat16)
            r = pltpu.make_async_remote_copy(
                src_ref=xbf_ref.at[rows],
                dst_ref=xrecv_ref.at[rows],
                send_sem=sem1s.at[c],
                recv_sem=sem1r.at[c],
                device_id=(1 - mx, my),
                device_id_type=pl.DeviceIdType.MESH,
            )
            r.start()
            rdma1.append(r)

        rdma2 = []
        for c in range(C):
            rows = pl.ds(c * mc, mc)
            rdma1[c].wait_recv()
            out_ref[rows, pl.ds(my * n, n)] = (
                xbf_ref[rows, :] + xrecv_ref[rows, :]
            )
            r = pltpu.make_async_remote_copy(
                src_ref=out_ref.at[rows, pl.ds(my * n, n)],
                dst_ref=out_ref.at[rows, pl.ds(my * n, n)],
                send_sem=sem2s.at[c],
                recv_sem=sem2r.at[c],
                device_id=(mx, 1 - my),
                device_id_type=pl.DeviceIdType.MESH,
            )
            r.start()
            rdma2.append(r)

        for c in range(C):
            rdma1[c].wait_send()
            rdma2[c].wait()

    return pl.pallas_call(
        body,
        out_shape=jax.ShapeDtypeStruct((m, 2 * n), jnp.bfloat16),
        in_specs=[pl.BlockSpec(memory_space=pltpu.VMEM)],
        out_specs=pl.BlockSpec(memory_space=pltpu.VMEM),
        scratch_shapes=[
            pltpu.VMEM((m, n), jnp.bfloat16),
            pltpu.VMEM((m, n), jnp.bfloat16),
            pltpu.SemaphoreType.DMA((C,)),
            pltpu.SemaphoreType.DMA((C,)),
            pltpu.SemaphoreType.DMA((C,)),
            pltpu.SemaphoreType.DMA((C,)),
        ],
        compiler_params=pltpu.CompilerParams(collective_id=0),
    )(x)
